# baseline (speedup 1.0000x reference)
import functools

import jax
import jax.numpy as jnp
from jax import lax
from jax.experimental import pallas as pl
from jax.experimental.pallas import tpu as pltpu
from jax.experimental.pallas import tpu_sc as plsc

N = 10000
D = 128
D2 = D // 2
G = 64
NPAD = 10240
BR = 1024
NB = NPAD // BR
NW = 32
CHUNK = 128
CS = 80
EPAD = NW * CS * CHUNK
FAST_CORE = 0
CA = 145
CB = 15
EA = 16 * CA * CHUNK
EB = 16 * CB * CHUNK
DEPTH = 5
RPS = NPAD // 16
RB = RPS // CHUNK


def _zero_vmem_f32(ref, nrows, ncols):
    zvec = jnp.zeros((16,), jnp.float32)
    npc = ncols // 16

    def body(i, _):
        ref[i // npc, pl.ds((i % npc) * 16, 16)] = zvec
        return 0

    lax.fori_loop(0, nrows * npc, body, 0)


@functools.cache
def _get_deg_kernel():
    mesh = plsc.VectorSubcoreMesh(core_axis_name="c", subcore_axis_name="s")
    return pl.kernel(
        _deg_body,
        mesh=mesh,
        out_type=jax.ShapeDtypeStruct((2, NPAD, D), jnp.float32),
        scratch_types=[
            pltpu.VMEM((CS, CHUNK), jnp.int32),
            pltpu.VMEM((CHUNK, D), jnp.float32),
            pltpu.VMEM_SHARED((NPAD, D), jnp.float32),
        ],
    )


def _deg_body(dst_hbm, out_hbm, dst_v, ones_v, deg_sh):
    cid = lax.axis_index("c")
    sid = lax.axis_index("s")
    wid = cid * 16 + sid
    pltpu.sync_copy(dst_hbm.at[wid], dst_v)
    _zero_vmem_f32(ones_v, CHUNK, D)
    for j in range(RB):
        pltpu.sync_copy(ones_v, deg_sh.at[pl.ds(sid * RPS + j * CHUNK, CHUNK)])
    ones16 = jnp.ones((16,), jnp.float32)
    npc = D // 16

    def fill(i, _):
        ones_v[i // npc, pl.ds((i % npc) * 16, 16)] = ones16
        return 0

    lax.fori_loop(0, CHUNK * npc, fill, 0)
    plsc.subcore_barrier()

    def body(ci, _):
        pltpu.sync_copy(ones_v, deg_sh.at[dst_v.at[ci]], add=True)
        return 0

    lax.fori_loop(0, CS, body, 0)
    plsc.subcore_barrier()
    for j in range(RB):
        r = sid * RPS + j * CHUNK
        pltpu.sync_copy(deg_sh.at[pl.ds(r, CHUNK)], ones_v)
        pltpu.sync_copy(ones_v, out_hbm.at[cid, pl.ds(r, CHUNK)])


@functools.cache
def _get_edge_scatter():
    mesh = plsc.VectorSubcoreMesh(core_axis_name="c", subcore_axis_name="s")
    return pl.kernel(
        _edge_scatter_body,
        mesh=mesh,
        out_type=(jax.ShapeDtypeStruct((2, NPAD, D2), jnp.float32),
                  jax.ShapeDtypeStruct((2, NPAD, D2), jnp.float32)),
        scratch_types=[
            pltpu.VMEM((CA, CHUNK), jnp.int32),
            pltpu.VMEM((CA, CHUNK), jnp.int32),
            pltpu.VMEM((CHUNK, D2), jnp.float32),
            pltpu.VMEM((CHUNK, D2), jnp.float32),
            pltpu.VMEM((CHUNK, D2), jnp.float32),
            pltpu.VMEM((CHUNK, D2), jnp.float32),
            pltpu.VMEM((CHUNK, D2), jnp.float32),
            pltpu.VMEM_SHARED((NPAD, D2), jnp.float32),
            pltpu.SemaphoreType.DMA,
            pltpu.SemaphoreType.DMA,
            pltpu.SemaphoreType.DMA,
            pltpu.SemaphoreType.DMA,
            pltpu.SemaphoreType.DMA,
        ],
        compiler_params=pltpu.CompilerParams(use_tc_tiling_on_sc=False),
    )


def _edge_scatter_body(glo_hbm, ghi_hbm, srca_hbm, dsta_hbm, srcb_hbm,
                       dstb_hbm, outlo_hbm, outhi_hbm, src_v, dst_v,
                       b0, b1, b2, b3, b4, acc_sh, s0, s1, s2, s3, s4):
    cid = lax.axis_index("c")
    sid = lax.axis_index("s")
    bufs = (b0, b1, b2, b3, b4)
    sems = (s0, s1, s2, s3, s4)

    @pl.when(cid == FAST_CORE)
    def _loada():
        pltpu.sync_copy(srca_hbm.at[sid], src_v)
        pltpu.sync_copy(dsta_hbm.at[sid], dst_v)

    @pl.when(cid != FAST_CORE)
    def _loadb():
        pltpu.sync_copy(srcb_hbm.at[sid], src_v.at[pl.ds(0, CB)])
        pltpu.sync_copy(dstb_hbm.at[sid], dst_v.at[pl.ds(0, CB)])

    def run_pass(g_hbm, out_hbm, nc):
        _zero_vmem_f32(b0, CHUNK, D2)
        for j in range(RB):
            pltpu.sync_copy(b0, acc_sh.at[pl.ds(sid * RPS + j * CHUNK, CHUNK)])
        plsc.subcore_barrier()
        for j in range(DEPTH - 1):
            pltpu.async_copy(g_hbm.at[src_v.at[j]], bufs[j], sems[j])

        def body(k, _):
            for j in range(DEPTH):
                c = DEPTH * k + j
                pltpu.make_async_copy(g_hbm.at[src_v.at[0]], bufs[j],
                                      sems[j]).wait()
                pltpu.sync_copy(bufs[j], acc_sh.at[dst_v.at[c]], add=True)
                nj = (j + DEPTH - 1) % DEPTH

                @pl.when(c + DEPTH - 1 < nc)
                def _next():
                    pltpu.async_copy(g_hbm.at[src_v.at[c + DEPTH - 1]],
                                     bufs[nj], sems[nj])
            return 0

        lax.fori_loop(0, nc // DEPTH, body, 0)
        plsc.subcore_barrier()
        for j in range(RB):
            r = sid * RPS + j * CHUNK
            pltpu.sync_copy(acc_sh.at[pl.ds(r, CHUNK)], b0)
            pltpu.sync_copy(b0, out_hbm.at[cid, pl.ds(r, CHUNK)])

    for g_hbm, out_hbm in ((glo_hbm, outlo_hbm), (ghi_hbm, outhi_hbm)):
        @pl.when(cid == FAST_CORE)
        def _fast():
            run_pass(g_hbm, out_hbm, CA)

        @pl.when(cid != FAST_CORE)
        def _slow():
            run_pass(g_hbm, out_hbm, CB)


def _first_body(x_ref, w_ref, dinv_ref, glo_ref, ghi_ref):
    res = jnp.dot(x_ref[...], w_ref[...],
                  preferred_element_type=jnp.float32) * dinv_ref[...]
    glo_ref[...] = res[:, :D2]
    ghi_ref[...] = res[:, D2:]


def _mid_body(alo_ref, ahi_ref, glo_ref, ghi_ref, dinv_ref, b_ref, w_ref,
              glo_out, ghi_out):
    lo = alo_ref[0] + alo_ref[1] + glo_ref[...]
    hi = ahi_ref[0] + ahi_ref[1] + ghi_ref[...]
    h = jnp.maximum(
        jnp.concatenate([lo, hi], axis=1) * dinv_ref[...] + b_ref[...], 0.0)
    res = jnp.dot(h, w_ref[...],
                  preferred_element_type=jnp.float32) * dinv_ref[...]
    glo_out[...] = res[:, :D2]
    ghi_out[...] = res[:, D2:]


def _pool_body(x_ref, alo_ref, ahi_ref, glo_ref, ghi_ref, dinv_ref, b_ref,
               batch_ref, lw_ref, lb_ref, out_ref, num_s, cnt_s):
    i = pl.program_id(0)

    @pl.when(i == 0)
    def _init():
        num_s[...] = jnp.zeros_like(num_s)
        cnt_s[...] = jnp.zeros_like(cnt_s)

    lo = alo_ref[0] + alo_ref[1] + glo_ref[...]
    hi = ahi_ref[0] + ahi_ref[1] + ghi_ref[...]
    h3 = jnp.maximum(
        jnp.concatenate([lo, hi], axis=1) * dinv_ref[...] + b_ref[...], 0.0)
    m = (jnp.sum(x_ref[...], axis=-1, keepdims=True) != 0)
    seg = lax.broadcasted_iota(jnp.int32, (BR, 128), 1)
    onehot = ((batch_ref[...] == seg) & m).astype(jnp.float32)
    dn = (((0,), (0,)), ((), ()))
    num_s[...] += lax.dot_general(onehot, h3, dn,
                                  preferred_element_type=jnp.float32)
    cnt_s[...] += lax.dot_general(onehot, jnp.ones((BR, 128), jnp.float32), dn,
                                  preferred_element_type=jnp.float32)

    @pl.when(i == NB - 1)
    def _fin():
        pooled = num_s[...] / jnp.maximum(cnt_s[...], 1.0)
        out_ref[...] = jnp.dot(pooled, lw_ref[...],
                               preferred_element_type=jnp.float32) + lb_ref[...]


def _row_spec(w):
    return pl.BlockSpec((BR, w), lambda i: (i, 0))


def _acc_spec():
    return pl.BlockSpec((2, BR, D2), lambda i: (0, i, 0))


def _full_spec(h, w):
    return pl.BlockSpec((h, w), lambda i: (0, 0))


def _first_tc(x, w, dinv):
    return pl.pallas_call(
        _first_body,
        grid=(NB,),
        in_specs=[_row_spec(D), _full_spec(D, D), _row_spec(1)],
        out_specs=(_row_spec(D2), _row_spec(D2)),
        out_shape=(jax.ShapeDtypeStruct((NPAD, D2), jnp.float32),
                   jax.ShapeDtypeStruct((NPAD, D2), jnp.float32)),
    )(x, w, dinv)


def _mid_tc(alo, ahi, glo, ghi, dinv, b, w):
    return pl.pallas_call(
        _mid_body,
        grid=(NB,),
        in_specs=[_acc_spec(), _acc_spec(), _row_spec(D2), _row_spec(D2),
                  _row_spec(1), _full_spec(1, D), _full_spec(D, D)],
        out_specs=(_row_spec(D2), _row_spec(D2)),
        out_shape=(jax.ShapeDtypeStruct((NPAD, D2), jnp.float32),
                   jax.ShapeDtypeStruct((NPAD, D2), jnp.float32)),
    )(alo, ahi, glo, ghi, dinv, b, w)


def _pool_tc(x, alo, ahi, glo, ghi, dinv, b, batch2d, lw, lb):
    return pl.pallas_call(
        _pool_body,
        grid=(NB,),
        in_specs=[_row_spec(D), _acc_spec(), _acc_spec(), _row_spec(D2),
                  _row_spec(D2), _row_spec(1), _full_spec(1, D), _row_spec(1),
                  _full_spec(D, 1), _full_spec(1, 1)],
        out_specs=_full_spec(128, 1),
        out_shape=jax.ShapeDtypeStruct((128, 1), jnp.float32),
        scratch_shapes=[pltpu.VMEM((128, 128), jnp.float32),
                        pltpu.VMEM((128, 128), jnp.float32)],
    )(x, alo, ahi, glo, ghi, dinv, b, batch2d, lw, lb)


def kernel(x, edge_index, batch, W0, b0, W1, b1, W2, b2, lin_W, lin_b):
    x = x.astype(jnp.float32)
    xp = jnp.pad(x, ((0, NPAD - N), (0, 0)))
    src = edge_index[0].astype(jnp.int32)
    dst = edge_index[1].astype(jnp.int32)
    E = src.shape[0]
    pad = EPAD - E
    src_p = jnp.concatenate([src, jnp.full((pad,), NPAD - 1, jnp.int32)])
    dst_p = jnp.concatenate([dst, jnp.full((pad,), NPAD - 1, jnp.int32)])
    dst_r = dst_p.reshape(NW, CS, CHUNK)
    src_a = src_p[:EA].reshape(16, CA, CHUNK)
    dst_a = dst_p[:EA].reshape(16, CA, CHUNK)
    src_b = src_p[EA:].reshape(16, CB, CHUNK)
    dst_b = dst_p[EA:].reshape(16, CB, CHUNK)
    batch2d = jnp.pad(batch.astype(jnp.int32), (0, NPAD - N),
                      constant_values=G).reshape(NPAD, 1)

    deg_parts = _get_deg_kernel()(dst_r)
    deg = deg_parts[0, :, 0] + deg_parts[1, :, 0] + 1.0
    dinv = lax.rsqrt(deg).reshape(NPAD, 1)

    b0r = b0.reshape(1, D)
    b1r = b1.reshape(1, D)
    b2r = b2.reshape(1, D)
    lbr = lin_b.reshape(1, 1)

    scatter = _get_edge_scatter()
    g0lo, g0hi = _first_tc(xp, W0, dinv)
    a0lo, a0hi = scatter(g0lo, g0hi, src_a, dst_a, src_b, dst_b)
    g1lo, g1hi = _mid_tc(a0lo, a0hi, g0lo, g0hi, dinv, b0r, W1)
    a1lo, a1hi = scatter(g1lo, g1hi, src_a, dst_a, src_b, dst_b)
    g2lo, g2hi = _mid_tc(a1lo, a1hi, g1lo, g1hi, dinv, b1r, W2)
    a2lo, a2hi = scatter(g2lo, g2hi, src_a, dst_a, src_b, dst_b)
    res = _pool_tc(xp, a2lo, a2hi, g2lo, g2hi, dinv, b2r, batch2d, lin_W, lbr)
    return res[:G, 0]


# speedup vs baseline: 1.0810x; 1.0810x over previous
import functools

import jax
import jax.numpy as jnp
from jax import lax
from jax.experimental import pallas as pl
from jax.experimental.pallas import tpu as pltpu
from jax.experimental.pallas import tpu_sc as plsc

N = 10000
D = 128
G = 64
NPAD = 10240
BR = 1024
NB = NPAD // BR
NW = 32
CHUNK = 128
C = 80
EPAD = NW * C * CHUNK
FAST_CORE = 0
CA = 144
CB = 16
CP = 16
EA = 16 * CA * CHUNK
EB = 16 * CB * CHUNK
RPS = NPAD // 16
RB = RPS // CHUNK

def _zero_vmem_f32(ref, nrows, ncols):
    zvec = jnp.zeros((16,), jnp.float32)
    npc = ncols // 16

    def body(i, _):
        r = i // npc
        cb = (i % npc) * 16
        ref[r, pl.ds(cb, 16)] = zvec
        return 0

    lax.fori_loop(0, nrows * npc, body, 0)


@functools.cache
def _get_deg_kernel():
    mesh = plsc.VectorSubcoreMesh(core_axis_name="c", subcore_axis_name="s")
    return pl.kernel(
        _deg_body,
        mesh=mesh,
        out_type=jax.ShapeDtypeStruct((2, NPAD, D), jnp.float32),
        scratch_types=[
            pltpu.VMEM((C, CHUNK), jnp.int32),
            pltpu.VMEM((CHUNK, D), jnp.float32),
            pltpu.VMEM_SHARED((NPAD, D), jnp.float32),
        ],
    )


def _deg_body(dst_hbm, out_hbm, dst_v, ones_v, deg_sh):
    cid = lax.axis_index("c")
    sid = lax.axis_index("s")
    wid = cid * 16 + sid
    pltpu.sync_copy(dst_hbm.at[wid], dst_v)
    _zero_vmem_f32(ones_v, CHUNK, D)
    for j in range(RB):
        pltpu.sync_copy(ones_v, deg_sh.at[pl.ds(sid * RPS + j * CHUNK, CHUNK)])
    ones16 = jnp.ones((16,), jnp.float32)
    npc = D // 16

    def fill(i, _):
        ones_v[i // npc, pl.ds((i % npc) * 16, 16)] = ones16
        return 0

    lax.fori_loop(0, CHUNK * npc, fill, 0)
    plsc.subcore_barrier()

    def body(ci, _):
        pltpu.sync_copy(ones_v, deg_sh.at[dst_v.at[ci]], add=True)
        return 0

    lax.fori_loop(0, C, body, 0)
    plsc.subcore_barrier()
    for j in range(RB):
        r = sid * RPS + j * CHUNK
        pltpu.sync_copy(deg_sh.at[pl.ds(r, CHUNK)], ones_v)
        pltpu.sync_copy(ones_v, out_hbm.at[cid, pl.ds(r, CHUNK)])


@functools.cache
def _get_edge_scatter():
    mesh = plsc.VectorSubcoreMesh(core_axis_name="c", subcore_axis_name="s")
    return pl.kernel(
        _edge_scatter_body,
        mesh=mesh,
        out_type=jax.ShapeDtypeStruct((2, NPAD, D), jnp.float32),
        scratch_types=[
            pltpu.VMEM((CP, CHUNK), jnp.int32),
            pltpu.VMEM((CP, CHUNK), jnp.int32),
            pltpu.VMEM((CHUNK, D), jnp.float32),
            pltpu.VMEM((CHUNK, D), jnp.float32),
            pltpu.VMEM_SHARED((NPAD, D), jnp.float32),
            pltpu.SemaphoreType.DMA,
            pltpu.SemaphoreType.DMA,
        ],
    )


def _edge_scatter_body(ga_hbm, srca_hbm, dsta_hbm, srcb_hbm, dstb_hbm,
                       out_hbm, src_v, dst_v, buf0_v, buf1_v, acc_sh,
                       sem0, sem1):
    cid = lax.axis_index("c")
    sid = lax.axis_index("s")

    def run_phase(src_hbm, dst_hbm, p):
        pltpu.sync_copy(src_hbm.at[sid, pl.ds(p * CP, CP)], src_v)
        pltpu.sync_copy(dst_hbm.at[sid, pl.ds(p * CP, CP)], dst_v)
        pltpu.async_copy(ga_hbm.at[src_v.at[0]], buf0_v, sem0)

        def body(k, _):
            c0 = 2 * k
            pltpu.async_copy(ga_hbm.at[src_v.at[c0 + 1]], buf1_v, sem1)
            pltpu.make_async_copy(ga_hbm.at[src_v.at[0]], buf0_v, sem0).wait()
            pltpu.sync_copy(buf0_v, acc_sh.at[dst_v.at[c0]], add=True)

            @pl.when(c0 + 2 < CP)
            def _next():
                pltpu.async_copy(ga_hbm.at[src_v.at[c0 + 2]], buf0_v, sem0)

            pltpu.make_async_copy(ga_hbm.at[src_v.at[0]], buf1_v, sem1).wait()
            pltpu.sync_copy(buf1_v, acc_sh.at[dst_v.at[c0 + 1]], add=True)
            return 0

        lax.fori_loop(0, CP // 2, body, 0)

    _zero_vmem_f32(buf0_v, CHUNK, D)
    for j in range(RB):
        pltpu.sync_copy(buf0_v,
                        acc_sh.at[pl.ds(sid * RPS + j * CHUNK, CHUNK)])
    plsc.subcore_barrier()

    @pl.when(cid == FAST_CORE)
    def _fast():
        for p in range(CA // CP):
            run_phase(srca_hbm, dsta_hbm, p)

    @pl.when(cid != FAST_CORE)
    def _slow():
        for p in range(CB // CP):
            run_phase(srcb_hbm, dstb_hbm, p)

    plsc.subcore_barrier()
    for j in range(RB):
        r = sid * RPS + j * CHUNK
        pltpu.sync_copy(acc_sh.at[pl.ds(r, CHUNK)], buf0_v)
        pltpu.sync_copy(buf0_v, out_hbm.at[cid, pl.ds(r, CHUNK)])


def _first_body(x_ref, w_ref, dinv_ref, g_ref):
    g_ref[...] = jnp.dot(x_ref[...], w_ref[...],
                         preferred_element_type=jnp.float32) * dinv_ref[...]


def _mid_body(a_ref, g_ref, dinv_ref, b_ref, w_ref, out_ref):
    h = jnp.maximum(
        (a_ref[0] + a_ref[1] + g_ref[...]) * dinv_ref[...] + b_ref[...],
        0.0)
    out_ref[...] = jnp.dot(h, w_ref[...],
                           preferred_element_type=jnp.float32) * dinv_ref[...]


def _pool_body(x_ref, a_ref, g_ref, dinv_ref, b_ref, batch_ref,
               lw_ref, lb_ref, out_ref, num_s, cnt_s):
    i = pl.program_id(0)

    @pl.when(i == 0)
    def _init():
        num_s[...] = jnp.zeros_like(num_s)
        cnt_s[...] = jnp.zeros_like(cnt_s)

    h3 = jnp.maximum(
        (a_ref[0] + a_ref[1] + g_ref[...]) * dinv_ref[...] + b_ref[...],
        0.0)
    m = (jnp.sum(x_ref[...], axis=-1, keepdims=True) != 0)
    seg = lax.broadcasted_iota(jnp.int32, (BR, 128), 1)
    onehot = ((batch_ref[...] == seg) & m).astype(jnp.float32)
    dn = (((0,), (0,)), ((), ()))
    num_s[...] += lax.dot_general(onehot, h3, dn,
                                  preferred_element_type=jnp.float32)
    cnt_s[...] += lax.dot_general(onehot, jnp.ones((BR, 128), jnp.float32), dn,
                                  preferred_element_type=jnp.float32)

    @pl.when(i == NB - 1)
    def _fin():
        pooled = num_s[...] / jnp.maximum(cnt_s[...], 1.0)
        out_ref[...] = jnp.dot(pooled, lw_ref[...],
                               preferred_element_type=jnp.float32) + lb_ref[...]


def _row_spec(w):
    return pl.BlockSpec((BR, w), lambda i: (i, 0))


def _acc_spec():
    return pl.BlockSpec((2, BR, D), lambda i: (0, i, 0))


def _full_spec(h, w):
    return pl.BlockSpec((h, w), lambda i: (0, 0))


def _first_tc(x, w, dinv):
    return pl.pallas_call(
        _first_body,
        grid=(NB,),
        in_specs=[_row_spec(D), _full_spec(D, D), _row_spec(1)],
        out_specs=_row_spec(D),
        out_shape=jax.ShapeDtypeStruct((NPAD, D), jnp.float32),
    )(x, w, dinv)


def _mid_tc(a, g, dinv, b, w):
    return pl.pallas_call(
        _mid_body,
        grid=(NB,),
        in_specs=[_acc_spec(), _row_spec(D), _row_spec(1),
                  _full_spec(1, D), _full_spec(D, D)],
        out_specs=_row_spec(D),
        out_shape=jax.ShapeDtypeStruct((NPAD, D), jnp.float32),
    )(a, g, dinv, b, w)


def _pool_tc(x, a, g, dinv, b, batch2d, lw, lb):
    return pl.pallas_call(
        _pool_body,
        grid=(NB,),
        in_specs=[_row_spec(D), _acc_spec(), _row_spec(D),
                  _row_spec(1), _full_spec(1, D), _row_spec(1),
                  _full_spec(D, 1), _full_spec(1, 1)],
        out_specs=_full_spec(128, 1),
        out_shape=jax.ShapeDtypeStruct((128, 1), jnp.float32),
        scratch_shapes=[pltpu.VMEM((128, 128), jnp.float32),
                        pltpu.VMEM((128, 128), jnp.float32)],
    )(x, a, g, dinv, b, batch2d, lw, lb)


def kernel(x, edge_index, batch, W0, b0, W1, b1, W2, b2, lin_W, lin_b):
    x = x.astype(jnp.float32)
    xp = jnp.pad(x, ((0, NPAD - N), (0, 0)))
    src = edge_index[0].astype(jnp.int32)
    dst = edge_index[1].astype(jnp.int32)
    E = src.shape[0]
    pad = EPAD - E
    src_p = jnp.concatenate([src, jnp.full((pad,), NPAD - 1, jnp.int32)])
    dst_p = jnp.concatenate([dst, jnp.full((pad,), NPAD - 1, jnp.int32)])
    src_r = src_p.reshape(NW, C, CHUNK)
    dst_r = dst_p.reshape(NW, C, CHUNK)
    src_a = src_p[:EA].reshape(16, CA, CHUNK)
    dst_a = dst_p[:EA].reshape(16, CA, CHUNK)
    src_b = src_p[EA:].reshape(16, CB, CHUNK)
    dst_b = dst_p[EA:].reshape(16, CB, CHUNK)
    batch2d = jnp.pad(batch.astype(jnp.int32), (0, NPAD - N),
                      constant_values=G).reshape(NPAD, 1)

    deg_parts = _get_deg_kernel()(dst_r)
    deg = deg_parts[0, :, 0] + deg_parts[1, :, 0] + 1.0
    dinv = lax.rsqrt(deg).reshape(NPAD, 1)

    b0r = b0.reshape(1, D)
    b1r = b1.reshape(1, D)
    b2r = b2.reshape(1, D)
    lbr = lin_b.reshape(1, 1)

    scatter = _get_edge_scatter()
    g0 = _first_tc(xp, W0, dinv)
    acc0 = scatter(g0, src_a, dst_a, src_b, dst_b)
    g1 = _mid_tc(acc0, g0, dinv, b0r, W1)
    acc1 = scatter(g1, src_a, dst_a, src_b, dst_b)
    g2 = _mid_tc(acc1, g1, dinv, b1r, W2)
    acc2 = scatter(g2, src_a, dst_a, src_b, dst_b)
    res = _pool_tc(xp, acc2, g2, dinv, b2r, batch2d, lin_W, lbr)
    return res[:G, 0]
